# Initial kernel scaffold; baseline (speedup 1.0000x reference)
import jax
import jax.numpy as jnp
from jax.experimental import pallas as pl


def kernel(x, edge_index, batch_index, grover_fp, W1, b1, W2, b2, Wm1, bm1, Wm2, bm2):
    raise NotImplementedError("write your pallas kernel here")


# speedup vs baseline: 11.2602x; 11.2602x over previous
import functools

import jax
import jax.numpy as jnp
from jax import lax
from jax.experimental import pallas as pl
from jax.experimental.pallas import tpu as pltpu
from jax.experimental.pallas import tpu_sc as plsc

N = 10000
E = 320000
NG = 256
D = 128
FP = 512
ODIM = 138
OPAD = 256

NC, NS = 2, 16
TILES = NC * NS
NPAD = 10240
CH = 128
NCH = 79
EPT = NCH * CH
EPAD = TILES * EPT
RPT = NPAD // NS
ZR = 64
DW = 16


HR = NPAD // D
HPT = HR // NS


def _deg_body(dst_hbm, iota_hbm, zer_hbm, out_hbm, acc_sh, hist_v, iota_v, zb, idx_v, sem):
    c = lax.axis_index("c")
    s = lax.axis_index("s")
    wid = c * NS + s
    pltpu.sync_copy(zer_hbm, hist_v)
    pltpu.sync_copy(iota_hbm, iota_v)

    @pl.when(s < HR // 8)
    def _():
        pltpu.sync_copy(zer_hbm.at[pl.ds(0, 8)], zb)
        pltpu.sync_copy(zb, acc_sh.at[pl.ds(s * 8, 8)])

    plsc.subcore_barrier()

    ones16 = jnp.ones((16,), jnp.float32)

    def step(j, carry):
        base = wid * EPT + j * CH
        pltpu.sync_copy(dst_hbm.at[pl.ds(base, CH)], idx_v)
        for k in range(CH // 16):
            idx = idx_v[pl.ds(k * 16, 16)]
            hi = lax.shift_right_logical(idx, 7)
            lo = jnp.bitwise_and(idx, 127)
            plsc.addupdate_scatter(hist_v, [hi, lo], ones16)
        return carry

    lax.fori_loop(0, NCH, step, 0)
    pltpu.sync_copy(hist_v, acc_sh.at[iota_v], add=True)
    plsc.subcore_barrier()

    @pl.when(s < HR // 8)
    def _():
        pltpu.sync_copy(acc_sh.at[pl.ds(s * 8, 8)], zb)
        pltpu.sync_copy(zb, out_hbm.at[c, pl.ds(s * 8, 8)])


@functools.lru_cache(maxsize=None)
def _deg_kernel():
    mesh = plsc.VectorSubcoreMesh(
        core_axis_name="c", subcore_axis_name="s", num_cores=NC, num_subcores=NS)
    return pl.kernel(
        _deg_body,
        out_type=jax.ShapeDtypeStruct((NC, HR, D), jnp.float32),
        mesh=mesh,
        compiler_params=pltpu.CompilerParams(needs_layout_passes=False),
        scratch_types=[
            pltpu.VMEM_SHARED((HR, D), jnp.float32),
            pltpu.VMEM((HR, D), jnp.float32),
            pltpu.VMEM((HR,), jnp.int32),
            pltpu.VMEM((8, D), jnp.float32),
            pltpu.VMEM((CH,), jnp.int32),
            pltpu.SemaphoreType.DMA,
        ],
    )


def _prop_body(src_hbm, dst_hbm, u_hbm, zer_hbm, out_hbm,
               acc_sh, rows_v, zb, isrc, idst, sem):
    c = lax.axis_index("c")
    s = lax.axis_index("s")
    wid = c * NS + s
    pltpu.sync_copy(zer_hbm, zb)
    for k in range(RPT // ZR):
        pltpu.sync_copy(zb, acc_sh.at[pl.ds(s * RPT + k * ZR, ZR)])
    plsc.subcore_barrier()

    def step(j, carry):
        base = wid * EPT + j * CH
        pltpu.sync_copy(src_hbm.at[pl.ds(base, CH)], isrc)
        pltpu.sync_copy(dst_hbm.at[pl.ds(base, CH)], idst)
        pltpu.async_copy(u_hbm.at[isrc], rows_v, sem).wait()
        pltpu.sync_copy(rows_v, acc_sh.at[idst], add=True)
        return carry

    lax.fori_loop(0, NCH, step, 0)
    plsc.subcore_barrier()
    for k in range(RPT // ZR):
        pltpu.sync_copy(acc_sh.at[pl.ds(s * RPT + k * ZR, ZR)], zb)
        pltpu.sync_copy(zb, out_hbm.at[c, pl.ds(s * RPT + k * ZR, ZR)])


@functools.lru_cache(maxsize=None)
def _prop_kernel():
    mesh = plsc.VectorSubcoreMesh(
        core_axis_name="c", subcore_axis_name="s", num_cores=NC, num_subcores=NS)
    return pl.kernel(
        _prop_body,
        out_type=jax.ShapeDtypeStruct((NC, NPAD, D), jnp.float32),
        mesh=mesh,
        scratch_types=[
            pltpu.VMEM_SHARED((NPAD, D), jnp.float32),
            pltpu.VMEM((CH, D), jnp.float32),
            pltpu.VMEM((ZR, D), jnp.float32),
            pltpu.VMEM((CH,), jnp.int32),
            pltpu.VMEM((CH,), jnp.int32),
            pltpu.SemaphoreType.DMA,
        ],
    )


BR = 512
GRID = NPAD // BR


def _uw_body(x_ref, w_ref, d0_ref, d1_ref, u_ref, dis_ref):
    pid = pl.program_id(0)
    row = pid * BR + lax.broadcasted_iota(jnp.int32, (BR, 1), 0)
    deg = d0_ref[...] + d1_ref[...] + 1.0
    dis = jnp.where(row < N, lax.rsqrt(deg), 0.0)
    h = jnp.dot(x_ref[...], w_ref[...], preferred_element_type=jnp.float32)
    u_ref[...] = h * dis
    dis_ref[...] = dis


def _tc_uw(xp, W1, d0, d1):
    return pl.pallas_call(
        _uw_body,
        grid=(GRID,),
        in_specs=[
            pl.BlockSpec((BR, D), lambda i: (i, 0)),
            pl.BlockSpec((D, D), lambda i: (0, 0)),
            pl.BlockSpec((BR, 1), lambda i: (i, 0)),
            pl.BlockSpec((BR, 1), lambda i: (i, 0)),
        ],
        out_specs=[
            pl.BlockSpec((BR, D), lambda i: (i, 0)),
            pl.BlockSpec((BR, 1), lambda i: (i, 0)),
        ],
        out_shape=[
            jax.ShapeDtypeStruct((NPAD, D), jnp.float32),
            jax.ShapeDtypeStruct((NPAD, 1), jnp.float32),
        ],
    )(xp, W1, d0, d1)


def _layer_body(a0_ref, a1_ref, u_ref, dis_ref, b_ref, w_ref, out_ref):
    dis = dis_ref[...]
    x1 = jnp.maximum(dis * (a0_ref[0] + a1_ref[0] + u_ref[...]) + b_ref[...], 0.0)
    out_ref[...] = jnp.dot(x1, w_ref[...], preferred_element_type=jnp.float32) * dis


def _tc_layer(acc, u, dis, b, W):
    return pl.pallas_call(
        _layer_body,
        grid=(GRID,),
        in_specs=[
            pl.BlockSpec((1, BR, D), lambda i: (0, i, 0)),
            pl.BlockSpec((1, BR, D), lambda i: (1, i, 0)),
            pl.BlockSpec((BR, D), lambda i: (i, 0)),
            pl.BlockSpec((BR, 1), lambda i: (i, 0)),
            pl.BlockSpec((1, D), lambda i: (0, 0)),
            pl.BlockSpec((D, D), lambda i: (0, 0)),
        ],
        out_specs=pl.BlockSpec((BR, D), lambda i: (i, 0)),
        out_shape=jax.ShapeDtypeStruct((NPAD, D), jnp.float32),
    )(acc, acc, u, dis, b, W)


def _pool_body(a0_ref, a1_ref, u_ref, dis_ref, b_ref, bat_ref, sum_ref, cnt_ref):
    pid = pl.program_id(0)
    dis = dis_ref[...]
    x2 = jnp.maximum(dis * (a0_ref[0] + a1_ref[0] + u_ref[...]) + b_ref[...], 0.0)
    sel = (bat_ref[...] == lax.broadcasted_iota(jnp.int32, (BR, NG), 1))
    s_mat = sel.astype(jnp.float32)

    @pl.when(pid == 0)
    def _():
        sum_ref[...] = jnp.zeros_like(sum_ref)
        cnt_ref[...] = jnp.zeros_like(cnt_ref)

    dims = (((0,), (0,)), ((), ()))
    sum_ref[...] += lax.dot_general(s_mat, x2, dims, preferred_element_type=jnp.float32)
    cnt_ref[...] += lax.dot_general(s_mat, jnp.ones((BR, D), jnp.float32), dims,
                                    preferred_element_type=jnp.float32)


def _tc_pool(acc, u, dis, b, batp):
    return pl.pallas_call(
        _pool_body,
        grid=(GRID,),
        in_specs=[
            pl.BlockSpec((1, BR, D), lambda i: (0, i, 0)),
            pl.BlockSpec((1, BR, D), lambda i: (1, i, 0)),
            pl.BlockSpec((BR, D), lambda i: (i, 0)),
            pl.BlockSpec((BR, 1), lambda i: (i, 0)),
            pl.BlockSpec((1, D), lambda i: (0, 0)),
            pl.BlockSpec((BR, 1), lambda i: (i, 0)),
        ],
        out_specs=[
            pl.BlockSpec((NG, D), lambda i: (0, 0)),
            pl.BlockSpec((NG, D), lambda i: (0, 0)),
        ],
        out_shape=[
            jax.ShapeDtypeStruct((NG, D), jnp.float32),
            jax.ShapeDtypeStruct((NG, D), jnp.float32),
        ],
    )(acc, acc, u, dis, b, batp)


def _mlp_body(sum_ref, cnt_ref, fp_ref, wa_ref, wb_ref, b1_ref, w2_ref, b2_ref, out_ref):
    gnn = sum_ref[...] / jnp.maximum(cnt_ref[...], 1.0)
    h = jnp.dot(gnn, wa_ref[...], preferred_element_type=jnp.float32)
    h += jnp.dot(fp_ref[...], wb_ref[...], preferred_element_type=jnp.float32)
    h = jnp.maximum(h + b1_ref[...], 0.0)
    out_ref[...] = jnp.dot(h, w2_ref[...], preferred_element_type=jnp.float32) + b2_ref[...]


def _tc_mlp(sums, cnts, fp, Wa, Wb, bm1, W2p, bm2p):
    return pl.pallas_call(
        _mlp_body,
        out_shape=jax.ShapeDtypeStruct((NG, OPAD), jnp.float32),
    )(sums, cnts, fp, Wa, Wb, bm1, W2p, bm2p)


def kernel(x, edge_index, batch_index, grover_fp, W1, b1, W2, b2, Wm1, bm1, Wm2, bm2):
    f32 = jnp.float32
    src = edge_index[0]
    dst = edge_index[1]
    epad = jnp.full((EPAD - E,), N, jnp.int32)
    srcp = jnp.concatenate([src, epad])
    dstp = jnp.concatenate([dst, epad])
    xp = jnp.pad(x, ((0, NPAD - N), (0, 0)))
    batp = jnp.pad(batch_index, (0, NPAD - N), constant_values=NG).reshape(NPAD, 1)

    iota80 = jnp.arange(HR, dtype=jnp.int32)
    zer80 = jnp.zeros((HR, D), f32)
    zerD = jnp.zeros((ZR, D), f32)

    degp = _deg_kernel()(dstp, iota80, zer80)
    d0 = degp[0].reshape(NPAD, 1)
    d1 = degp[1].reshape(NPAD, 1)

    u1, dis = _tc_uw(xp, W1, d0, d1)
    acc1 = _prop_kernel()(srcp, dstp, u1, zerD)
    u2 = _tc_layer(acc1, u1, dis, b1.reshape(1, D), W2)
    acc2 = _prop_kernel()(srcp, dstp, u2, zerD)
    sums, cnts = _tc_pool(acc2, u2, dis, b2.reshape(1, D), batp)

    W2p = jnp.zeros((D, OPAD), f32).at[:, :ODIM].set(Wm2)
    b2p = jnp.zeros((1, OPAD), f32).at[0, :ODIM].set(bm2)
    out = _tc_mlp(sums, cnts, grover_fp, Wm1[:D], Wm1[D:],
                  bm1.reshape(1, D), W2p, b2p)
    return out[:, :ODIM]
